# Initial kernel scaffold; baseline (speedup 1.0000x reference)
import jax
import jax.numpy as jnp
from jax.experimental import pallas as pl


def kernel(pos, batch, params):
    raise NotImplementedError("write your pallas kernel here")


# speedup vs baseline: 1.0000x; 1.0000x over previous
import jax
import jax.numpy as jnp
from jax.experimental import pallas as pl

R = 0.2
MAX_K = 20
H1, C1 = 9, 27
H2, C2 = 9, 64
BN_EPS = 1e-5
NUM_CLASSES = 40


def _pairwise_d2(xb, x):
    d2 = (xb * xb).sum(1)[:, None] + (x * x).sum(1)[None, :] - 2.0 * jnp.matmul(
        xb, x.T, precision=jax.lax.Precision.HIGHEST)
    return jnp.maximum(d2, 0.0)


def _radius(x, batch, r, max_k):
    N = x.shape[0]
    rows, cols, keeps = [], [], []
    chunk = 2500
    k = min(max_k, N)
    for s in range(0, N, chunk):
        xb = x[s:s + chunk]
        d2 = _pairwise_d2(xb, x)
        valid = (d2 <= r * r) & (batch[s:s + xb.shape[0], None] == batch[None, :])
        score = jnp.where(valid, -d2, -jnp.inf)
        vals, idx = jax.lax.top_k(score, k)
        keep = vals > -jnp.inf
        q = jnp.broadcast_to(jnp.arange(s, s + xb.shape[0])[:, None], idx.shape)
        rows.append(q.reshape(-1))
        cols.append(idx.reshape(-1))
        keeps.append(keep.reshape(-1))
    return jnp.concatenate(rows), jnp.concatenate(cols), jnp.concatenate(keeps)


def _build_edges(x, batch, r=R, max_k=MAX_K):
    row, col, keep = _radius(x, batch, r, max_k)
    ei0, ei1 = col, row
    keep = keep & (ei0 != ei1)
    N = x.shape[0]
    mx = jnp.max(jnp.where(keep, jnp.maximum(ei0, ei1), -1))
    num_nodes = mx + 1
    loop = jnp.arange(N)
    loop_keep = loop < num_nodes
    ei0 = jnp.concatenate([ei0, loop])
    ei1 = jnp.concatenate([ei1, loop])
    keep = jnp.concatenate([keep, loop_keep])
    ei0 = jnp.where(keep, ei0, N).astype(jnp.int32)
    ei1 = jnp.where(keep, ei1, 0).astype(jnp.int32)
    return ei0, ei1


def _att_conv(x, ei, ej, W, lw, lb, heads, out_c):
    N = x.shape[0]
    xw = x @ W
    xi = xw[jnp.minimum(ei, N - 1)].reshape(-1, heads, out_c)
    xj = xw[ej].reshape(-1, heads, out_c)
    scores = jnp.einsum('ehc,egc->eh', xi, xj) / jnp.sqrt(jnp.asarray(out_c, jnp.float32))
    m = jax.ops.segment_max(scores, ei, num_segments=N + 1)
    m = jnp.where(jnp.isfinite(m), m, 0.0)
    e = jnp.exp(scores - m[ei])
    s = jax.ops.segment_sum(e, ei, num_segments=N + 1)
    alpha = e / (s[ei] + 1e-16)
    msg = (xj * alpha[:, :, None]).reshape(-1, heads * out_c)
    agg = jax.ops.segment_max(msg, ei, num_segments=N + 1)
    agg = jnp.where(jnp.isfinite(agg), agg, 0.0)
    return agg[:N] @ lw + lb


def _bn_eval(x, g, b):
    return g * x / jnp.sqrt(1.0 + BN_EPS) + b


def _head_pallas(g, p):
    def body(g_ref, m1w, m1b, mg1, mb1, m2w, m2b, mg2, mb2, ow, ob, out_ref):
        h = jnp.maximum(jnp.dot(g_ref[...], m1w[...]) + m1b[...], 0.0)
        h = _bn_eval(h, mg1[...], mb1[...])
        h = jnp.maximum(jnp.dot(h, m2w[...]) + m2b[...], 0.0)
        h = _bn_eval(h, mg2[...], mb2[...])
        logits = jnp.dot(h, ow[...]) + ob[...]
        mx = jnp.max(logits, axis=1, keepdims=True)
        lse = jnp.log(jnp.sum(jnp.exp(logits - mx), axis=1, keepdims=True)) + mx
        out_ref[...] = logits - lse

    args = (g, p['m1_w'], p['m1_b'].reshape(1, -1), p['mbn1_g'].reshape(1, -1),
            p['mbn1_b'].reshape(1, -1), p['m2_w'], p['m2_b'].reshape(1, -1),
            p['mbn2_g'].reshape(1, -1), p['mbn2_b'].reshape(1, -1),
            p['out_w'], p['out_b'].reshape(1, -1))
    return pl.pallas_call(
        body,
        out_shape=jax.ShapeDtypeStruct((g.shape[0], NUM_CLASSES), jnp.float32),
    )(*args)


def kernel(pos, batch, params):
    e1 = _build_edges(pos, batch)
    x1 = _att_conv(pos, e1[0], e1[1], params['W1'], params['lin1_w'], params['lin1_b'], H1, C1)
    e2 = _build_edges(x1, batch)
    x2 = _att_conv(x1, e2[0], e2[1], params['W2'], params['lin2_w'], params['lin2_b'], H2, C2)
    h = jnp.concatenate([x1, x2], axis=1)
    h = _bn_eval(jax.nn.relu(h @ params['fc1_w'] + params['fc1_b']), params['bn1_g'], params['bn1_b'])
    G = 8
    g = jax.ops.segment_max(h, batch, num_segments=G)
    g = jnp.where(jnp.isfinite(g), g, 0.0)
    return _head_pallas(g, params)


# speedup vs baseline: 1.7254x; 1.7253x over previous
import functools

import jax
import jax.numpy as jnp
from jax.experimental import pallas as pl
from jax.experimental.pallas import tpu as pltpu

R = 0.2
MAX_K = 20
H1, C1 = 9, 27
H2, C2 = 9, 64
BN_EPS = 1e-5
NUM_CLASSES = 40
NP_PAD = 10240
BLK = 128


def _topk_body(xT_ref, nsqT_ref, batT_ref, xb_ref, batb_ref, nbr_ref, rowmx_ref,
               score_ref, *, r2, n_real):
    blk = xb_ref.shape[0]
    npad = xT_ref.shape[1]
    i0 = pl.program_id(0)
    xb = xb_ref[...]
    mm = jax.lax.dot_general(
        xb, xT_ref[...], (((1,), (0,)), ((), ())),
        precision=jax.lax.Precision.HIGHEST, preferred_element_type=jnp.float32)
    nb = jnp.sum(xb * xb, axis=1, keepdims=True)
    d2 = jnp.maximum(nb + nsqT_ref[...] - 2.0 * mm, 0.0)
    valid = (d2 <= r2) & (batb_ref[...] == batT_ref[...])
    score_ref[...] = jnp.where(valid, -d2, -jnp.inf)

    colid = jax.lax.broadcasted_iota(jnp.int32, (blk, npad), 1)
    qvec = i0 * blk + jax.lax.broadcasted_iota(jnp.int32, (blk, 1), 0)

    rowmx = jnp.full((blk, 1), -1, jnp.int32)
    for k in range(MAX_K):
        s = score_ref[...]
        m = jnp.max(s, axis=1, keepdims=True)
        found = m > -jnp.inf
        cand = jnp.where(s == m, colid, npad)
        idx = jnp.min(cand, axis=1, keepdims=True)
        score_ref[...] = jnp.where(colid == idx, -jnp.inf, s)
        keep = found & (idx != qvec)
        nbr_ref[:, k:k + 1] = jnp.where(keep, idx, n_real)
        rowmx = jnp.maximum(rowmx, jnp.where(keep, jnp.maximum(idx, qvec), -1))
    nbr_ref[:, MAX_K:] = jnp.full((blk, 32 - MAX_K), n_real, jnp.int32)
    rowmx_ref[...] = rowmx


def _radius_topk(x, batch, n_real):
    N, d = x.shape
    xp = jnp.zeros((NP_PAD, d), jnp.float32).at[:N].set(x)
    batp = jnp.full((NP_PAD,), -1, jnp.int32) - jnp.arange(NP_PAD, dtype=jnp.int32)
    batp = batp.at[:N].set(batch.astype(jnp.int32))
    xT = xp.T
    nsqT = jnp.sum(xT * xT, axis=0, keepdims=True)
    batT = batp.reshape(1, NP_PAD)
    batb = batp.reshape(NP_PAD, 1)

    grid = NP_PAD // BLK
    body = functools.partial(_topk_body, r2=R * R, n_real=n_real)
    nbr, rowmx = pl.pallas_call(
        body,
        grid=(grid,),
        in_specs=[
            pl.BlockSpec((d, NP_PAD), lambda i: (0, 0)),
            pl.BlockSpec((1, NP_PAD), lambda i: (0, 0)),
            pl.BlockSpec((1, NP_PAD), lambda i: (0, 0)),
            pl.BlockSpec((BLK, d), lambda i: (i, 0)),
            pl.BlockSpec((BLK, 1), lambda i: (i, 0)),
        ],
        out_specs=[
            pl.BlockSpec((BLK, 32), lambda i: (i, 0)),
            pl.BlockSpec((BLK, 1), lambda i: (i, 0)),
        ],
        out_shape=[
            jax.ShapeDtypeStruct((NP_PAD, 32), jnp.int32),
            jax.ShapeDtypeStruct((NP_PAD, 1), jnp.int32),
        ],
        scratch_shapes=[pltpu.VMEM((BLK, NP_PAD), jnp.float32)],
    )(xT, nsqT, batT, xp, batb)
    return nbr, rowmx


def _build_edges_pallas(x, batch):
    N = x.shape[0]
    nbr, rowmx = _radius_topk(x, batch, N)
    nbr = nbr[:N, :MAX_K]
    kept = nbr != N
    q = jnp.broadcast_to(jnp.arange(N, dtype=jnp.int32)[:, None], nbr.shape)
    ei0 = nbr.reshape(-1)
    ei1 = jnp.where(kept, q, 0).reshape(-1)
    num_nodes = jnp.max(rowmx) + 1
    loop = jnp.arange(N, dtype=jnp.int32)
    loop_keep = loop < num_nodes
    ei0 = jnp.concatenate([ei0, jnp.where(loop_keep, loop, N)])
    ei1 = jnp.concatenate([ei1, jnp.where(loop_keep, loop, 0)])
    return ei0, ei1


def _att_conv(x, ei, ej, W, lw, lb, heads, out_c):
    N = x.shape[0]
    xw = x @ W
    xi = xw[jnp.minimum(ei, N - 1)].reshape(-1, heads, out_c)
    xj = xw[ej].reshape(-1, heads, out_c)
    scores = jnp.einsum('ehc,egc->eh', xi, xj) / jnp.sqrt(jnp.asarray(out_c, jnp.float32))
    m = jax.ops.segment_max(scores, ei, num_segments=N + 1)
    m = jnp.where(jnp.isfinite(m), m, 0.0)
    e = jnp.exp(scores - m[ei])
    s = jax.ops.segment_sum(e, ei, num_segments=N + 1)
    alpha = e / (s[ei] + 1e-16)
    msg = (xj * alpha[:, :, None]).reshape(-1, heads * out_c)
    agg = jax.ops.segment_max(msg, ei, num_segments=N + 1)
    agg = jnp.where(jnp.isfinite(agg), agg, 0.0)
    return agg[:N] @ lw + lb


def _bn_eval(x, g, b):
    return g * x / jnp.sqrt(1.0 + BN_EPS) + b


def _head_pallas(g, p):
    def body(g_ref, m1w, m1b, mg1, mb1, m2w, m2b, mg2, mb2, ow, ob, out_ref):
        h = jnp.maximum(jnp.dot(g_ref[...], m1w[...]) + m1b[...], 0.0)
        h = _bn_eval(h, mg1[...], mb1[...])
        h = jnp.maximum(jnp.dot(h, m2w[...]) + m2b[...], 0.0)
        h = _bn_eval(h, mg2[...], mb2[...])
        logits = jnp.dot(h, ow[...]) + ob[...]
        mx = jnp.max(logits, axis=1, keepdims=True)
        lse = jnp.log(jnp.sum(jnp.exp(logits - mx), axis=1, keepdims=True)) + mx
        out_ref[...] = logits - lse

    args = (g, p['m1_w'], p['m1_b'].reshape(1, -1), p['mbn1_g'].reshape(1, -1),
            p['mbn1_b'].reshape(1, -1), p['m2_w'], p['m2_b'].reshape(1, -1),
            p['mbn2_g'].reshape(1, -1), p['mbn2_b'].reshape(1, -1),
            p['out_w'], p['out_b'].reshape(1, -1))
    return pl.pallas_call(
        body,
        out_shape=jax.ShapeDtypeStruct((g.shape[0], NUM_CLASSES), jnp.float32),
    )(*args)


def kernel(pos, batch, params):
    e1 = _build_edges_pallas(pos, batch)
    x1 = _att_conv(pos, e1[0], e1[1], params['W1'], params['lin1_w'], params['lin1_b'], H1, C1)
    e2 = _build_edges_pallas(x1, batch)
    x2 = _att_conv(x1, e2[0], e2[1], params['W2'], params['lin2_w'], params['lin2_b'], H2, C2)
    h = jnp.concatenate([x1, x2], axis=1)
    h = _bn_eval(jax.nn.relu(h @ params['fc1_w'] + params['fc1_b']), params['bn1_g'], params['bn1_b'])
    G = 8
    g = jax.ops.segment_max(h, batch, num_segments=G)
    g = jnp.where(jnp.isfinite(g), g, 0.0)
    return _head_pallas(g, params)
